# Initial kernel scaffold; baseline (speedup 1.0000x reference)
import jax
import jax.numpy as jnp
from jax.experimental import pallas as pl


def kernel(x, pos, conv1_w, conv1_b, conv2_w, conv2_b, conv3_w, conv3_b, conv4_w, conv4_b, conv5_w, conv5_b, conv6_w, conv6_b):
    raise NotImplementedError("write your pallas kernel here")


# speedup vs baseline: 1.9210x; 1.9210x over previous
import jax
import jax.numpy as jnp
from jax.experimental import pallas as pl
from jax.experimental.pallas import tpu as pltpu


def _zero_halo(ref, H, W, pad):
    C = ref.shape[-1]
    Wp = W + 2 * pad
    zrow = jnp.zeros((pad, Wp, C), ref.dtype)
    zcol = jnp.zeros((H, pad, C), ref.dtype)
    ref[0:pad, 0:Wp, :] = zrow
    ref[pad + H:pad + H + pad, 0:Wp, :] = zrow
    ref[pad:pad + H, 0:pad, :] = zcol
    ref[pad:pad + H, pad + W:pad + W + pad, :] = zcol


def _convt_up(xp_ref, w_ref, b_ref, out_ref, H, W, out_pad, row_chunk):
    C4 = w_ref.shape[-1]
    C = C4 // 4
    bias = b_ref[0]
    for h0 in range(0, H, row_chunk):
        hc = min(row_chunk, H - h0)
        acc = jnp.zeros((hc, W, C4), jnp.float32)
        for t in range(3):
            for s in range(3):
                acc = acc + jax.lax.dot_general(
                    xp_ref[h0 + 2 - t:h0 + 2 - t + hc, 2 - s:2 - s + W, :],
                    w_ref[t, s],
                    dimension_numbers=(((2,), (0,)), ((), ())),
                    preferred_element_type=jnp.float32)
        acc = jnp.maximum(acc + bias, 0.0)
        a0 = acc[:, :, 0 * C:1 * C]
        a1 = acc[:, :, 1 * C:2 * C]
        a2 = acc[:, :, 2 * C:3 * C]
        a3 = acc[:, :, 3 * C:4 * C]
        r0 = jnp.stack([a0, a1], axis=2).reshape(hc, 2 * W, C)
        r1 = jnp.stack([a2, a3], axis=2).reshape(hc, 2 * W, C)
        blk = jnp.stack([r0, r1], axis=1).reshape(2 * hc, 2 * W, C)
        out_ref[out_pad + 2 * h0:out_pad + 2 * h0 + 2 * hc,
                out_pad:out_pad + 2 * W, :] = blk


def _decoder_kernel(z_ref, pos_ref, w1, b1, w2, b2, w3, b3, w4, b4,
                    w5, b5, w6, b6, o_ref, c1, c2, c3, c4, c5, c6):
    H0, W0 = pos_ref.shape[1], pos_ref.shape[2]
    H1, W1 = 2 * H0, 2 * W0
    H2, W2 = 4 * H0, 4 * W0
    H3, W3 = 8 * H0, 8 * W0
    H4, W4 = 16 * H0, 16 * W0

    _zero_halo(c1, H0, W0, 1)
    c1[1:1 + H0, 1:1 + W0, :] = pos_ref[0] + z_ref[0, 0][None, None, :]

    _zero_halo(c2, H1, W1, 1)
    _convt_up(c1, w1, b1, c2, H0, W0, 1, H0)
    _zero_halo(c3, H2, W2, 1)
    _convt_up(c2, w2, b2, c3, H1, W1, 1, 8)
    _zero_halo(c4, H3, W3, 1)
    _convt_up(c3, w3, b3, c4, H2, W2, 1, 8)
    _zero_halo(c5, H4, W4, 2)
    _convt_up(c4, w4, b4, c5, H3, W3, 2, 8)

    _zero_halo(c6, H4, W4, 1)
    C5 = w5.shape[-1]
    b5v = b5[0]
    for h0 in range(0, H4, 16):
        hc = min(16, H4 - h0)
        acc = jnp.zeros((hc, W4, C5), jnp.float32)
        for ky in range(5):
            for kx in range(5):
                acc = acc + jax.lax.dot_general(
                    c5[h0 + ky:h0 + ky + hc, kx:kx + W4, :], w5[ky, kx],
                    dimension_numbers=(((2,), (0,)), ((), ())),
                    preferred_element_type=jnp.float32)
        c6[1 + h0:1 + h0 + hc, 1:1 + W4, :] = jnp.maximum(acc + b5v, 0.0)

    C6 = w6.shape[-1]
    b6v = b6[0]
    for h0 in range(0, H4, 16):
        hc = min(16, H4 - h0)
        acc = jnp.zeros((hc, W4, C6), jnp.float32)
        for ky in range(3):
            for kx in range(3):
                acc = acc + jax.lax.dot_general(
                    c6[h0 + ky:h0 + ky + hc, kx:kx + W4, :], w6[ky, kx],
                    dimension_numbers=(((2,), (0,)), ((), ())),
                    preferred_element_type=jnp.float32)
        o_ref[0, h0:h0 + hc] = acc + b6v


def kernel(x, pos, conv1_w, conv1_b, conv2_w, conv2_b, conv3_w, conv3_b,
           conv4_w, conv4_b, conv5_w, conv5_b, conv6_w, conv6_b):
    D = x.shape[-1]
    z = x.reshape(-1, D)
    N = z.shape[0]
    H0, W0 = pos.shape[1], pos.shape[2]
    HF, WF = 16 * H0, 16 * W0
    C4 = conv1_w.shape[-1]
    C5 = conv5_w.shape[-1]
    C6 = conv6_w.shape[-1]

    const4 = lambda n: (0, 0, 0, 0)
    const2 = lambda n: (0, 0)
    in_specs = [
        pl.BlockSpec((1, 1, D), lambda n: (n, 0, 0)),
        pl.BlockSpec((1, H0, W0, D), const4),
        pl.BlockSpec(conv1_w.shape, const4), pl.BlockSpec((1, C4), const2),
        pl.BlockSpec(conv2_w.shape, const4), pl.BlockSpec((1, C4), const2),
        pl.BlockSpec(conv3_w.shape, const4), pl.BlockSpec((1, C4), const2),
        pl.BlockSpec(conv4_w.shape, const4), pl.BlockSpec((1, C4), const2),
        pl.BlockSpec(conv5_w.shape, const4), pl.BlockSpec((1, C5), const2),
        pl.BlockSpec(conv6_w.shape, const4), pl.BlockSpec((1, C6), const2),
    ]
    scratch = [
        pltpu.VMEM((H0 + 2, W0 + 2, D), jnp.float32),
        pltpu.VMEM((2 * H0 + 2, 2 * W0 + 2, D), jnp.float32),
        pltpu.VMEM((4 * H0 + 2, 4 * W0 + 2, D), jnp.float32),
        pltpu.VMEM((8 * H0 + 2, 8 * W0 + 2, D), jnp.float32),
        pltpu.VMEM((HF + 4, WF + 4, D), jnp.float32),
        pltpu.VMEM((HF + 2, WF + 2, C5), jnp.float32),
    ]
    return pl.pallas_call(
        _decoder_kernel,
        out_shape=jax.ShapeDtypeStruct((N, HF, WF, C6), jnp.float32),
        grid_spec=pltpu.PrefetchScalarGridSpec(
            num_scalar_prefetch=0,
            grid=(N,),
            in_specs=in_specs,
            out_specs=pl.BlockSpec((1, HF, WF, C6), lambda n: (n, 0, 0, 0)),
            scratch_shapes=scratch,
        ),
        compiler_params=pltpu.CompilerParams(
            dimension_semantics=("parallel",),
            vmem_limit_bytes=64 * 1024 * 1024),
    )(z.reshape(N, 1, D), pos, conv1_w, conv1_b.reshape(1, C4),
      conv2_w, conv2_b.reshape(1, C4), conv3_w, conv3_b.reshape(1, C4),
      conv4_w, conv4_b.reshape(1, C4), conv5_w, conv5_b.reshape(1, C5),
      conv6_w, conv6_b.reshape(1, C6))
